# baseline (speedup 1.0000x reference)
import functools

import jax
import jax.numpy as jnp
from jax import lax
from jax.experimental import pallas as pl
from jax.experimental.pallas import tpu as pltpu
from jax.experimental.pallas import tpu_sc as plsc

BATCH = 16384
EMBED_DIM = 32
NUM_CORES = 2
NUM_SUBCORES = 16
NUM_WORKERS = NUM_CORES * NUM_SUBCORES
B_PER_W = BATCH // NUM_WORKERS
LANES = 16
GROUPS = B_PER_W // LANES


def _sc_body(u_ids_hbm, i_ids_hbm, ue_hbm, ie_hbm, ub_hbm, ib_hbm, bias_hbm,
             out_hbm,
             uidx_v, iidx_v, urows_v, irows_v, ub_v, ib_v, bias_v, out_v,
             sem):
    wid = lax.axis_index("s") * NUM_CORES + lax.axis_index("c")
    base = wid * B_PER_W

    pltpu.sync_copy(u_ids_hbm.at[pl.ds(base, B_PER_W)], uidx_v)
    pltpu.sync_copy(i_ids_hbm.at[pl.ds(base, B_PER_W)], iidx_v)
    pltpu.sync_copy(bias_hbm, bias_v)

    c_ue = pltpu.make_async_copy(ue_hbm.at[uidx_v], urows_v, sem)
    c_ie = pltpu.make_async_copy(ie_hbm.at[iidx_v], irows_v, sem)
    c_ub = pltpu.make_async_copy(ub_hbm.at[uidx_v], ub_v, sem)
    c_ib = pltpu.make_async_copy(ib_hbm.at[iidx_v], ib_v, sem)
    c_ue.start()
    c_ie.start()
    c_ub.start()
    c_ib.start()
    c_ue.wait()
    c_ie.wait()
    c_ub.wait()
    c_ib.wait()

    bias_vec = bias_v[...]

    def group(g, carry):
        rows = g * LANES + lax.iota(jnp.int32, LANES)
        acc = bias_vec + ub_v[pl.ds(g * LANES, LANES)] + ib_v[pl.ds(g * LANES, LANES)]
        for d in range(EMBED_DIM):
            cols = jnp.full((LANES,), d, dtype=jnp.int32)
            u = plsc.load_gather(urows_v, [rows, cols])
            i = plsc.load_gather(irows_v, [rows, cols])
            acc = acc + u * i
        out_v[pl.ds(g * LANES, LANES)] = acc
        return carry

    lax.fori_loop(0, GROUPS, group, 0)

    pltpu.sync_copy(out_v, out_hbm.at[pl.ds(base, B_PER_W)])


@jax.jit
def kernel(u_ids, i_ids, user_embeddings, item_embeddings, user_bias,
           item_bias, bias):
    bias16 = jnp.broadcast_to(bias.astype(jnp.float32), (LANES,))
    mesh = plsc.VectorSubcoreMesh(core_axis_name="c", subcore_axis_name="s",
                                  num_cores=NUM_CORES)
    f = pl.kernel(
        _sc_body,
        out_type=jax.ShapeDtypeStruct((BATCH,), jnp.float32),
        mesh=mesh,
        compiler_params=pltpu.CompilerParams(use_tc_tiling_on_sc=False,
                                             needs_layout_passes=False),
        scratch_types=[
            pltpu.VMEM((B_PER_W,), jnp.int32),
            pltpu.VMEM((B_PER_W,), jnp.int32),
            pltpu.VMEM((B_PER_W, EMBED_DIM), jnp.float32),
            pltpu.VMEM((B_PER_W, EMBED_DIM), jnp.float32),
            pltpu.VMEM((B_PER_W,), jnp.float32),
            pltpu.VMEM((B_PER_W,), jnp.float32),
            pltpu.VMEM((LANES,), jnp.float32),
            pltpu.VMEM((B_PER_W,), jnp.float32),
            pltpu.SemaphoreType.DMA,
        ],
    )
    return f(u_ids, i_ids, user_embeddings, item_embeddings, user_bias,
             item_bias, bias16)


# speedup vs baseline: 3.7391x; 3.7391x over previous
import jax
import jax.numpy as jnp
from jax import lax
from jax.experimental import pallas as pl
from jax.experimental.pallas import tpu as pltpu
from jax.experimental.pallas import tpu_sc as plsc

BATCH = 16384
EMBED_DIM = 32
NUM_CORES = 2
NUM_SUBCORES = 16
NUM_WORKERS = NUM_CORES * NUM_SUBCORES
B_PER_W = BATCH // NUM_WORKERS
LANES = 16
ROUNDS = B_PER_W // LANES
SLAB = 16
PASSES = EMBED_DIM // SLAB


def _sc_body(u_ids_hbm, i_ids_hbm, ue_t_hbm, ie_t_hbm, ub_hbm, ib_hbm,
             bias_hbm, out_hbm,
             uidx_v, iidx_v, ubuf_v, ibuf_v, ub_v, ib_v,
             bias_v, out_v, sem):
    wid = lax.axis_index("s") * NUM_CORES + lax.axis_index("c")
    base = wid * B_PER_W

    pltpu.sync_copy(u_ids_hbm.at[pl.ds(base, B_PER_W)], uidx_v)
    pltpu.sync_copy(i_ids_hbm.at[pl.ds(base, B_PER_W)], iidx_v)
    pltpu.sync_copy(bias_hbm, bias_v)

    cb = pltpu.make_async_copy(ub_hbm.at[uidx_v], ub_v, sem)
    ci = pltpu.make_async_copy(ib_hbm.at[iidx_v], ib_v, sem)
    cb.start()
    ci.start()
    cb.wait()
    ci.wait()

    bias_vec = bias_v[...]
    dvec = lax.iota(jnp.int32, LANES)
    svec_base = lax.iota(jnp.int32, LANES)

    def rnd(r, carry):
        off = r * LANES
        uvec = uidx_v[pl.ds(off, LANES)]
        ivec = iidx_v[pl.ds(off, LANES)]
        ulan = jnp.bitwise_and(uvec, 127)
        ilan = jnp.bitwise_and(ivec, 127)
        ublk = jnp.bitwise_and(uvec, -128)
        iblk = jnp.bitwise_and(ivec, -128)
        acc0 = bias_vec + ub_v[pl.ds(off, LANES)] + ib_v[pl.ds(off, LANES)]

        def one_pass(p, acc):
            drow = p * SLAB

            for s in range(LANES):
                uo = pl.multiple_of(ublk[s], 128)
                io = pl.multiple_of(iblk[s], 128)
                pltpu.make_async_copy(
                    ue_t_hbm.at[pl.ds(drow, SLAB), pl.ds(uo, 128)],
                    ubuf_v.at[s], sem).start()
                pltpu.make_async_copy(
                    ie_t_hbm.at[pl.ds(drow, SLAB), pl.ds(io, 128)],
                    ibuf_v.at[s], sem).start()

            def drain(s, c2):
                pltpu.make_async_copy(
                    ue_t_hbm.at[pl.ds(0, SLAB), pl.ds(0, 128)],
                    ubuf_v.at[s], sem).wait()
                pltpu.make_async_copy(
                    ie_t_hbm.at[pl.ds(0, SLAB), pl.ds(0, 128)],
                    ibuf_v.at[s], sem).wait()
                return c2

            lax.fori_loop(0, LANES, drain, 0)

            def dim(d, a):
                dv = jnp.full((LANES,), d, jnp.int32)
                gu = plsc.load_gather(ubuf_v, [svec_base, dv, ulan])
                gi = plsc.load_gather(ibuf_v, [svec_base, dv, ilan])
                return a + gu * gi

            return lax.fori_loop(0, SLAB, dim, acc)

        acc = lax.fori_loop(0, PASSES, one_pass, acc0)
        out_v[pl.ds(off, LANES)] = acc
        return carry

    lax.fori_loop(0, ROUNDS, rnd, 0)

    pltpu.sync_copy(out_v, out_hbm.at[pl.ds(base, B_PER_W)])


@jax.jit
def kernel(u_ids, i_ids, user_embeddings, item_embeddings, user_bias,
           item_bias, bias):
    bias16 = jnp.broadcast_to(bias.astype(jnp.float32), (LANES,))
    mesh = plsc.VectorSubcoreMesh(core_axis_name="c", subcore_axis_name="s",
                                  num_cores=NUM_CORES)
    f = pl.kernel(
        _sc_body,
        out_type=jax.ShapeDtypeStruct((BATCH,), jnp.float32),
        mesh=mesh,
        compiler_params=pltpu.CompilerParams(needs_layout_passes=False),
        scratch_types=[
            pltpu.VMEM((B_PER_W,), jnp.int32),
            pltpu.VMEM((B_PER_W,), jnp.int32),
            pltpu.VMEM((LANES, SLAB, 128), jnp.float32),
            pltpu.VMEM((LANES, SLAB, 128), jnp.float32),
            pltpu.VMEM((B_PER_W,), jnp.float32),
            pltpu.VMEM((B_PER_W,), jnp.float32),
            pltpu.VMEM((LANES,), jnp.float32),
            pltpu.VMEM((B_PER_W,), jnp.float32),
            pltpu.SemaphoreType.DMA,
        ],
    )
    return f(u_ids, i_ids, user_embeddings.T, item_embeddings.T, user_bias,
             item_bias, bias16)
